# baseline (device time: 10155 ns/iter reference)
import jax
import jax.numpy as jnp
from jax import lax
from jax.experimental import pallas as pl
from jax.experimental.pallas import tpu as pltpu

N_DEV = 4
EPS = 1e-5


def kernel(x, gamma, beta):
    m, n_loc = x.shape
    n_global = n_loc * N_DEV

    def body(x_ref, g_ref, b_ref, out_ref, gather_ref, send_sems, recv_sems):
        my = lax.axis_index("i")

        barrier = pltpu.get_barrier_semaphore()
        for off in (1, 2, 3):
            pl.semaphore_signal(
                barrier, inc=1,
                device_id=((my + off) % N_DEV,),
                device_id_type=pl.DeviceIdType.MESH,
            )
        pl.semaphore_wait(barrier, N_DEV - 1)

        ident = (
            lax.broadcasted_iota(jnp.int32, (m, m), 0)
            == lax.broadcasted_iota(jnp.int32, (m, m), 1)
        ).astype(jnp.float32)

        x = x_ref[:, :]
        s = jnp.sum(x, axis=1, keepdims=True)
        sq = jnp.sum(x * x, axis=1, keepdims=True)
        packed = jnp.concatenate([s, sq], axis=1)
        gather_ref[0, :, :] = lax.dot_general(
            packed, ident,
            dimension_numbers=(((0,), (0,)), ((), ())),
            precision=lax.Precision.HIGHEST,
        )

        rdmas = []
        for off in (1, 2, 3):
            rdma = pltpu.make_async_remote_copy(
                src_ref=gather_ref.at[0],
                dst_ref=gather_ref.at[off],
                send_sem=send_sems.at[off],
                recv_sem=recv_sems.at[off],
                device_id=((my + off) % N_DEV,),
                device_id_type=pl.DeviceIdType.MESH,
            )
            rdma.start()
            rdmas.append(rdma)
        for rdma in rdmas:
            rdma.wait()

        tot = (
            gather_ref[0, :, :] + gather_ref[1, :, :]
            + gather_ref[2, :, :] + gather_ref[3, :, :]
        )
        tot_cols = lax.dot_general(
            ident, tot,
            dimension_numbers=(((1,), (1,)), ((), ())),
            precision=lax.Precision.HIGHEST,
        )
        mean = tot_cols[:, 0:1] / n_global
        var = tot_cols[:, 1:2] / n_global - mean * mean
        inv = lax.rsqrt(var + EPS)
        g = g_ref[:].reshape(1, n_loc)
        b = b_ref[:].reshape(1, n_loc)
        out_ref[:, :] = g * ((x - mean) * inv) + b

    return pl.pallas_call(
        body,
        out_shape=jax.ShapeDtypeStruct((m, n_loc), jnp.float32),
        in_specs=[
            pl.BlockSpec(memory_space=pltpu.VMEM),
            pl.BlockSpec(memory_space=pltpu.VMEM),
            pl.BlockSpec(memory_space=pltpu.VMEM),
        ],
        out_specs=pl.BlockSpec(memory_space=pltpu.VMEM),
        scratch_shapes=[
            pltpu.VMEM((N_DEV, 2, m), jnp.float32),
            pltpu.SemaphoreType.DMA((N_DEV,)),
            pltpu.SemaphoreType.DMA((N_DEV,)),
        ],
        compiler_params=pltpu.CompilerParams(collective_id=0),
    )(x, gamma, beta)


# device time: 7725 ns/iter; 1.3146x vs baseline; 1.3146x over previous
import jax
import jax.numpy as jnp
from jax import lax
from jax.experimental import pallas as pl
from jax.experimental.pallas import tpu as pltpu

N_DEV = 4
EPS = 1e-5


def kernel(x, gamma, beta):
    m, n_loc = x.shape
    n_global = n_loc * N_DEV

    def body(x_ref, g_ref, b_ref, i_ref, out_ref, gather_ref,
             send_sems, recv_sems):
        my = lax.axis_index("i")

        barrier = pltpu.get_barrier_semaphore()
        for off in (1, 2, 3):
            pl.semaphore_signal(
                barrier, inc=1,
                device_id=((my + off) % N_DEV,),
                device_id_type=pl.DeviceIdType.MESH,
            )
        pl.semaphore_wait(barrier, N_DEV - 1)

        ident = i_ref[:, :]

        x = x_ref[:, :]
        s = jnp.sum(x, axis=1, keepdims=True)
        sq = jnp.sum(x * x, axis=1, keepdims=True)
        packed = jnp.concatenate([s, sq], axis=1)
        gather_ref[0, :, :] = lax.dot_general(
            packed, ident,
            dimension_numbers=(((0,), (0,)), ((), ())),
        )

        rdmas = []
        for off in (1, 2, 3):
            rdma = pltpu.make_async_remote_copy(
                src_ref=gather_ref.at[0],
                dst_ref=gather_ref.at[off],
                send_sem=send_sems.at[off],
                recv_sem=recv_sems.at[off],
                device_id=((my + off) % N_DEV,),
                device_id_type=pl.DeviceIdType.MESH,
            )
            rdma.start()
            rdmas.append(rdma)
        for rdma in rdmas:
            rdma.wait()

        tot = (
            gather_ref[0, :, :] + gather_ref[1, :, :]
            + gather_ref[2, :, :] + gather_ref[3, :, :]
        )
        tot_cols = lax.dot_general(
            ident, tot,
            dimension_numbers=(((1,), (1,)), ((), ())),
        )
        mean = tot_cols[:, 0:1] / n_global
        var = tot_cols[:, 1:2] / n_global - mean * mean
        inv = lax.rsqrt(var + EPS)
        g = g_ref[:].reshape(1, n_loc)
        b = b_ref[:].reshape(1, n_loc)
        out_ref[:, :] = g * ((x - mean) * inv) + b

    ident = jnp.eye(m, dtype=jnp.float32)
    return pl.pallas_call(
        body,
        out_shape=jax.ShapeDtypeStruct((m, n_loc), jnp.float32),
        in_specs=[
            pl.BlockSpec(memory_space=pltpu.VMEM),
            pl.BlockSpec(memory_space=pltpu.VMEM),
            pl.BlockSpec(memory_space=pltpu.VMEM),
            pl.BlockSpec(memory_space=pltpu.VMEM),
        ],
        out_specs=pl.BlockSpec(memory_space=pltpu.VMEM),
        scratch_shapes=[
            pltpu.VMEM((N_DEV, 2, m), jnp.float32),
            pltpu.SemaphoreType.DMA((N_DEV,)),
            pltpu.SemaphoreType.DMA((N_DEV,)),
        ],
        compiler_params=pltpu.CompilerParams(collective_id=0),
    )(x, gamma, beta, ident)
